# baseline (speedup 1.0000x reference)
import jax
import jax.numpy as jnp
from jax import lax
from jax.experimental import pallas as pl
from jax.experimental.pallas import tpu as pltpu
from jax.experimental.pallas import tpu_sc as plsc

B, S, D = 4096, 200, 128
N = B * S
NC, NS = 2, 16
NW = NC * NS
C = 128
CON_ROWS = 1001
EX_ROWS = 100001

PIECES = 2
NP = N // PIECES
PER_W = NP // NW
CHUNKS = PER_W // C


def _sc_body(e_idx, c_idx, ex_t, con_t, diff_t, q_out, pid_out,
             eidx_v, cidx_v, exb, conb, pidb, con_sh,
             sem_e, sem_c, sem_p, sem_wb):
    cid = lax.axis_index("c")
    sid = lax.axis_index("s")
    wid = sid * NC + cid
    base_w = wid * PER_W

    @pl.when(sid == 0)
    def _():
        pltpu.sync_copy(con_t, con_sh)

    pltpu.sync_copy(e_idx.at[pl.ds(base_w, PER_W)], eidx_v)
    pltpu.sync_copy(c_idx.at[pl.ds(base_w, PER_W)], cidx_v)
    plsc.subcore_barrier()

    def fire_gathers(i, b):
        off = i * C
        es = eidx_v.at[pl.ds(off, C)]
        cs = cidx_v.at[pl.ds(off, C)]
        pltpu.async_copy(ex_t.at[es], exb.at[b], sem_e.at[b])
        pltpu.async_copy(con_sh.at[cs], conb.at[b], sem_c.at[b])
        pltpu.async_copy(diff_t.at[es], pidb.at[b], sem_p.at[b])

    def wait_gathers(i, b):
        off = i * C
        es = eidx_v.at[pl.ds(off, C)]
        cs = cidx_v.at[pl.ds(off, C)]
        pltpu.make_async_copy(ex_t.at[es], exb.at[b], sem_e.at[b]).wait()
        pltpu.make_async_copy(con_sh.at[cs], conb.at[b], sem_c.at[b]).wait()
        pltpu.make_async_copy(diff_t.at[es], pidb.at[b], sem_p.at[b]).wait()

    def fire_wb(i, b):
        base = base_w + i * C
        pltpu.async_copy(conb.at[b], q_out.at[pl.ds(base, C)], sem_wb.at[b])
        pltpu.async_copy(pidb.at[b], pid_out.at[pl.ds(base, C)], sem_wb.at[b])

    def wait_wb(b):
        pltpu.make_async_copy(conb.at[b], q_out.at[pl.ds(base_w, C)],
                              sem_wb.at[b]).wait()
        pltpu.make_async_copy(pidb.at[b], pid_out.at[pl.ds(base_w, C)],
                              sem_wb.at[b]).wait()

    def combine(b):
        def group(g, _):
            for k in range(16):
                t = g * 16 + k
                for j in range(D // 16):
                    sl = (b, t, pl.ds(j * 16, 16))
                    conb[sl] = conb[sl] + exb[sl]
            return 0

        lax.fori_loop(0, C // 16, group, 0)

    fire_gathers(0, 0)

    def pair(p, _):
        for b in range(2):
            i = 2 * p + b
            wait_gathers(i, b)
            if b == 0:
                @pl.when(p > 0)
                def _():
                    wait_wb(1)
                fire_gathers(i + 1, 1)
            else:
                wait_wb(0)

                @pl.when(p < CHUNKS // 2 - 1)
                def _():
                    fire_gathers(i + 1, 0)
            combine(b)
            fire_wb(i, b)
        return 0

    lax.fori_loop(0, CHUNKS // 2, pair, 0)
    wait_wb(1)


def _sc_gather_combine(e_piece, c_piece, ex_t, con_t, diff_flat):
    mesh = plsc.VectorSubcoreMesh(core_axis_name="c", subcore_axis_name="s",
                                  num_cores=NC, num_subcores=NS)
    f = pl.kernel(
        _sc_body,
        out_type=[jax.ShapeDtypeStruct((NP, D), jnp.float32),
                  jax.ShapeDtypeStruct((NP,), jnp.float32)],
        mesh=mesh,
        scratch_types=[
            pltpu.VMEM((PER_W,), jnp.int32),
            pltpu.VMEM((PER_W,), jnp.int32),
            pltpu.VMEM((2, C, D), jnp.float32),
            pltpu.VMEM((2, C, D), jnp.float32),
            pltpu.VMEM((2, C), jnp.float32),
            pltpu.VMEM_SHARED((CON_ROWS, D), jnp.float32),
            pltpu.SemaphoreType.DMA((2,)),
            pltpu.SemaphoreType.DMA((2,)),
            pltpu.SemaphoreType.DMA((2,)),
            pltpu.SemaphoreType.DMA((2,)),
        ],
    )
    return f(e_piece, c_piece, ex_t, con_t, diff_flat)


RT = 8192


def _scale_body(ex_ref, df_ref, out_ref):
    out_ref[...] = ex_ref[...] * df_ref[...]


def _scale_table(ex_t, diff_t):
    grid = ((EX_ROWS + RT - 1) // RT,)
    return pl.pallas_call(
        _scale_body,
        grid=grid,
        in_specs=[
            pl.BlockSpec((RT, D), lambda i: (i, 0)),
            pl.BlockSpec((RT, 1), lambda i: (i, 0)),
        ],
        out_specs=pl.BlockSpec((RT, D), lambda i: (i, 0)),
        out_shape=jax.ShapeDtypeStruct((EX_ROWS, D), jnp.float32),
    )(ex_t, diff_t)


R = 8192
PIECE_BLOCKS = NP // R


def _tc_body_first(q_ref, m_ref, w1t_ref, row0_ref, diff_ref,
                   qfull_ref, qafull_ref):
    acc = jax.lax.dot_general(
        q_ref[...], w1t_ref[...], (((1,), (0,)), ((), ())),
        preferred_element_type=jnp.float32,
        precision=jax.lax.Precision.HIGHEST)
    qafull_ref[...] = acc + row0_ref[...] + m_ref[...] * diff_ref[...]
    qfull_ref[...] = q_ref[...]


def _tc_body_rest(q_ref, m_ref, w1t_ref, row0_ref, diff_ref,
                  qin_ref, qain_ref, qfull_ref, qafull_ref):
    del qin_ref, qain_ref
    _tc_body_first(q_ref, m_ref, w1t_ref, row0_ref, diff_ref,
                   qfull_ref, qafull_ref)


def _tc_linear_piece(k, q_piece, respf_piece, w1t, row0, diff, carry):
    off = k * PIECE_BLOCKS
    in_specs = [
        pl.BlockSpec((R, D), lambda i: (i, 0)),
        pl.BlockSpec((R, 1), lambda i: (i, 0)),
        pl.BlockSpec((D, D), lambda i: (0, 0)),
        pl.BlockSpec((1, D), lambda i: (0, 0)),
        pl.BlockSpec((1, D), lambda i: (0, 0)),
    ]
    args = [q_piece, respf_piece, w1t, row0, diff]
    if k == 0:
        body = _tc_body_first
        aliases = {}
    else:
        body = _tc_body_rest
        in_specs += [pl.BlockSpec(memory_space=pl.ANY),
                     pl.BlockSpec(memory_space=pl.ANY)]
        args += list(carry)
        aliases = {5: 0, 6: 1}
    return pl.pallas_call(
        body,
        grid=(PIECE_BLOCKS,),
        in_specs=in_specs,
        out_specs=[pl.BlockSpec((R, D), lambda i: (i + off, 0)),
                   pl.BlockSpec((R, D), lambda i: (i + off, 0))],
        out_shape=[jax.ShapeDtypeStruct((N, D), jnp.float32),
                   jax.ShapeDtypeStruct((N, D), jnp.float32)],
        input_output_aliases=aliases,
    )(*args)


def kernel(exercise_seq, concept_seq, response_seq, exercise_table,
           concept_table, difficult_table, a_table, W, b):
    e_flat = exercise_seq.reshape(-1).astype(jnp.int32)
    c_flat = concept_seq.reshape(-1).astype(jnp.int32)
    diff_flat = difficult_table.reshape(-1)

    scaled_table = _scale_table(exercise_table, difficult_table)

    w1t = W[:, :D].T
    w2t = W[:, D:].T
    rows = a_table @ w2t + b[None, :]
    row0 = rows[0:1, :]
    diffrow = rows[1:2, :] - row0
    respf = response_seq.reshape(-1, 1).astype(jnp.float32)

    q_pieces = []
    pid_pieces = []
    for k in range(PIECES):
        qk, pk = _sc_gather_combine(
            e_flat[k * NP:(k + 1) * NP], c_flat[k * NP:(k + 1) * NP],
            scaled_table, concept_table, diff_flat)
        q_pieces.append(qk)
        pid_pieces.append(pk)

    carry = None
    for k in range(PIECES):
        carry = _tc_linear_piece(
            k, q_pieces[k], respf[k * NP:(k + 1) * NP], w1t, row0, diffrow,
            carry)
    q_flat, qa_flat = carry

    pid_flat = jnp.concatenate(pid_pieces)

    q = q_flat.reshape(B, S, D)
    qa = qa_flat.reshape(B, S, D)
    pid = pid_flat.reshape(B, S, 1)
    return (q, qa, pid)


# speedup vs baseline: 1.3813x; 1.3813x over previous
import functools

import jax
import jax.numpy as jnp
from jax import lax
from jax.experimental import pallas as pl
from jax.experimental.pallas import tpu as pltpu
from jax.experimental.pallas import tpu_sc as plsc

B, S, D = 4096, 200, 128
N = B * S
NC, NS = 2, 16
NW = NC * NS
PER_W = N // NW
C = 128
CHUNKS = PER_W // C
CON_ROWS = 1001


def _sc_body(e_idx, c_idx, ex_t, con_t, diff_t, q_out, pid_out,
             eidx_v, cidx_v, exb, conb, pidb, con_sh,
             sem_e, sem_c, sem_p, sem_wb):
    cid = lax.axis_index("c")
    sid = lax.axis_index("s")
    wid = sid * NC + cid
    base_w = wid * PER_W

    @pl.when(sid == 0)
    def _():
        pltpu.sync_copy(con_t, con_sh)

    pltpu.sync_copy(e_idx.at[pl.ds(base_w, PER_W)], eidx_v)
    pltpu.sync_copy(c_idx.at[pl.ds(base_w, PER_W)], cidx_v)
    plsc.subcore_barrier()

    def fire_gathers(i, b):
        off = i * C
        es = eidx_v.at[pl.ds(off, C)]
        cs = cidx_v.at[pl.ds(off, C)]
        pltpu.async_copy(ex_t.at[es], exb.at[b], sem_e.at[b])
        pltpu.async_copy(con_sh.at[cs], conb.at[b], sem_c.at[b])
        pltpu.async_copy(diff_t.at[es], pidb.at[b], sem_p.at[b])

    def wait_gathers(i, b):
        off = i * C
        es = eidx_v.at[pl.ds(off, C)]
        cs = cidx_v.at[pl.ds(off, C)]
        pltpu.make_async_copy(ex_t.at[es], exb.at[b], sem_e.at[b]).wait()
        pltpu.make_async_copy(con_sh.at[cs], conb.at[b], sem_c.at[b]).wait()
        pltpu.make_async_copy(diff_t.at[es], pidb.at[b], sem_p.at[b]).wait()

    def fire_wb(i, b):
        base = base_w + i * C
        pltpu.async_copy(conb.at[b], q_out.at[pl.ds(base, C)], sem_wb.at[b])
        pltpu.async_copy(pidb.at[b], pid_out.at[pl.ds(base, C)], sem_wb.at[b])

    def wait_wb(b):
        pltpu.make_async_copy(conb.at[b], q_out.at[pl.ds(base_w, C)],
                              sem_wb.at[b]).wait()
        pltpu.make_async_copy(pidb.at[b], pid_out.at[pl.ds(base_w, C)],
                              sem_wb.at[b]).wait()

    def combine(b):
        @plsc.parallel_loop(0, C, step=4, unroll=2)
        def _(t0):
            for k in range(4):
                t = t0 + k
                for j in range(D // 16):
                    sl = (b, t, pl.ds(j * 16, 16))
                    conb[sl] = conb[sl] + exb[sl]

    fire_gathers(0, 0)

    def pair(p, _):
        for b in range(2):
            i = 2 * p + b
            wait_gathers(i, b)
            if b == 0:
                @pl.when(p > 0)
                def _():
                    wait_wb(1)
                fire_gathers(i + 1, 1)
            else:
                wait_wb(0)

                @pl.when(p < CHUNKS // 2 - 1)
                def _():
                    fire_gathers(i + 1, 0)
            combine(b)
            fire_wb(i, b)
        return 0

    lax.fori_loop(0, CHUNKS // 2, pair, 0)
    wait_wb(1)


def _sc_gather_combine(e_flat, c_flat, ex_t, con_t, diff_flat):
    mesh = plsc.VectorSubcoreMesh(core_axis_name="c", subcore_axis_name="s",
                                  num_cores=NC, num_subcores=NS)
    f = pl.kernel(
        _sc_body,
        out_type=[jax.ShapeDtypeStruct((N, D), jnp.float32),
                  jax.ShapeDtypeStruct((N,), jnp.float32)],
        mesh=mesh,
        scratch_types=[
            pltpu.VMEM((PER_W,), jnp.int32),
            pltpu.VMEM((PER_W,), jnp.int32),
            pltpu.VMEM((2, C, D), jnp.float32),
            pltpu.VMEM((2, C, D), jnp.float32),
            pltpu.VMEM((2, C), jnp.float32),
            pltpu.VMEM_SHARED((CON_ROWS, D), jnp.float32),
            pltpu.SemaphoreType.DMA((2,)),
            pltpu.SemaphoreType.DMA((2,)),
            pltpu.SemaphoreType.DMA((2,)),
            pltpu.SemaphoreType.DMA((2,)),
        ],
    )
    return f(e_flat, c_flat, ex_t, con_t, diff_flat)


EX_ROWS = 100001
RT = 8192


def _scale_body(ex_ref, df_ref, out_ref):
    out_ref[...] = ex_ref[...] * df_ref[...]


def _scale_table(ex_t, diff_t):
    grid = ((EX_ROWS + RT - 1) // RT,)
    return pl.pallas_call(
        _scale_body,
        grid=grid,
        in_specs=[
            pl.BlockSpec((RT, D), lambda i: (i, 0)),
            pl.BlockSpec((RT, 1), lambda i: (i, 0)),
        ],
        out_specs=pl.BlockSpec((RT, D), lambda i: (i, 0)),
        out_shape=jax.ShapeDtypeStruct((EX_ROWS, D), jnp.float32),
    )(ex_t, diff_t)


R = 8192


def _tc_body(q_ref, m_ref, w1t_ref, row0_ref, diff_ref, out_ref):
    acc = jax.lax.dot_general(
        q_ref[...], w1t_ref[...], (((1,), (0,)), ((), ())),
        preferred_element_type=jnp.float32,
        precision=jax.lax.Precision.HIGHEST)
    out_ref[...] = acc + row0_ref[...] + m_ref[...] * diff_ref[...]


def _tc_linear(q, respf, w1t, row0, diff):
    grid = (N // R,)
    return pl.pallas_call(
        _tc_body,
        grid=grid,
        in_specs=[
            pl.BlockSpec((R, D), lambda i: (i, 0)),
            pl.BlockSpec((R, 1), lambda i: (i, 0)),
            pl.BlockSpec((D, D), lambda i: (0, 0)),
            pl.BlockSpec((1, D), lambda i: (0, 0)),
            pl.BlockSpec((1, D), lambda i: (0, 0)),
        ],
        out_specs=pl.BlockSpec((R, D), lambda i: (i, 0)),
        out_shape=jax.ShapeDtypeStruct((N, D), jnp.float32),
    )(q, respf, w1t, row0, diff)


def kernel(exercise_seq, concept_seq, response_seq, exercise_table,
           concept_table, difficult_table, a_table, W, b):
    e_flat = exercise_seq.reshape(-1).astype(jnp.int32)
    c_flat = concept_seq.reshape(-1).astype(jnp.int32)
    diff_flat = difficult_table.reshape(-1)

    scaled_table = _scale_table(exercise_table, difficult_table)
    q_flat, pid_flat = _sc_gather_combine(
        e_flat, c_flat, scaled_table, concept_table, diff_flat)

    w1t = W[:, :D].T
    w2t = W[:, D:].T
    rows = a_table @ w2t + b[None, :]
    row0 = rows[0:1, :]
    diff = rows[1:2, :] - row0
    respf = response_seq.reshape(-1, 1).astype(jnp.float32)

    qa_flat = _tc_linear(q_flat, respf, w1t, row0, diff)

    q = q_flat.reshape(B, S, D)
    qa = qa_flat.reshape(B, S, D)
    pid = pid_flat.reshape(B, S, 1)
    return (q, qa, pid)
